# baseline (speedup 1.0000x reference)
import functools

import jax
import jax.numpy as jnp
from jax import lax
from jax.experimental import pallas as pl
from jax.experimental.pallas import tpu as pltpu
from jax.experimental.pallas import tpu_sc as plsc

_B = 16384
_D = 128
_ARITY = 6
_OUT = 16
_NW = 32
_RPW = _B // _NW
_L = 16
_CHUNKS = _RPW // _L
_NBLK = 4
_RPB = _RPW // _NBLK

_TWO_OVER_PI = 0.6366197723675814
_RND = 12582912.0
_PIO2_A = 1.5707963705062866
_PIO2_B = 4.37113900018624e-08


def _sincos_core(v, qoff):
    t = v * _TWO_OVER_PI
    kf = (t + _RND) - _RND
    r = (v - kf * _PIO2_A) + kf * _PIO2_B
    ki = kf.astype(jnp.int32) + qoff
    r2 = r * r
    sinp = r * (1.0 + r2 * (-1.6666667e-1 + r2 * (8.3333310e-3 + r2 * (-1.9840874e-4))))
    cosp = 1.0 + r2 * (-5.0e-1 + r2 * (4.1666638e-2 + r2 * (-1.3887316e-3)))
    p = jnp.where((ki & 1) == 1, cosp, sinp)
    return jnp.where((ki & 2) == 2, -p, p)


def _fast_sin(v):
    return _sincos_core(v, 0)


def _fast_cos(v):
    return _sincos_core(v, 1)


def _bases(g):
    return [_fast_sin(g[0]), _fast_cos(g[1]), g[2] * g[3], g[4] + g[5]]


def _sel4(c, h):
    return jnp.where(c == 0, h[0],
                     jnp.where(c == 1, h[1],
                               jnp.where(c == 2, h[2], h[3])))


@functools.cache
def _build():
    @functools.partial(
        pl.kernel,
        mesh=plsc.VectorSubcoreMesh(core_axis_name="c", subcore_axis_name="s"),
        compiler_params=pltpu.CompilerParams(
            needs_layout_passes=False, use_tc_tiling_on_sc=True),
        out_type=jax.ShapeDtypeStruct((_OUT, _B), jnp.float32),
        scratch_types=[
            pltpu.VMEM((_RPW, _D), jnp.float32),
            pltpu.VMEM((_ARITY, _RPW), jnp.int32),
            pltpu.VMEM((_ARITY, _RPW), jnp.int32),
            pltpu.VMEM((_OUT, _RPW), jnp.int32),
            pltpu.VMEM((_OUT, _RPW), jnp.float32),
            pltpu.SemaphoreType.DMA,
        ] + [pltpu.SemaphoreType.DMA] * _NBLK,
    )
    def _occam_sc(x_hbm, i1t_hbm, i2t_hbm, i3t_hbm, out_hbm,
                  x_v, i1_v, i2_v, i3_v, o_v, sem_i, *sem_x):
        wid = lax.axis_index("s") * 2 + lax.axis_index("c")
        base = wid * _RPW
        xcopies = [
            pltpu.async_copy(
                x_hbm.at[pl.ds(base + blk * _RPB, _RPB)],
                x_v.at[pl.ds(blk * _RPB, _RPB)], sem_x[blk])
            for blk in range(_NBLK)
        ]
        c1 = pltpu.async_copy(i1t_hbm.at[:, pl.ds(base, _RPW)], i1_v, sem_i)
        c2 = pltpu.async_copy(i2t_hbm.at[:, pl.ds(base, _RPW)], i2_v, sem_i)
        c3 = pltpu.async_copy(i3t_hbm.at[:, pl.ds(base, _RPW)], i3_v, sem_i)
        c1.wait(); c2.wait(); c3.wait()

        lanes = lax.iota(jnp.int32, _L)

        def chunk(c, carry):
            rows = lanes + c * _L
            cols = pl.ds(c * _L, _L)
            g1 = [plsc.load_gather(x_v, [rows, i1_v[j, cols]])
                  for j in range(_ARITY)]
            h1 = _bases(g1)
            g2 = [_sel4(i2_v[j, cols], h1) for j in range(_ARITY)]
            h2 = _bases(g2)
            for o in range(_OUT):
                o_v[o, cols] = _sel4(i3_v[o, cols], h2)
            return carry

        cpb = _RPB // _L
        for blk in range(_NBLK):
            xcopies[blk].wait()
            lax.fori_loop(blk * cpb, (blk + 1) * cpb, chunk, 0)
        pltpu.sync_copy(o_v, out_hbm.at[:, pl.ds(base, _RPW)])

    return _occam_sc


def kernel(x, W1, W2, W3, idx1, idx2, idx3):
    del W1, W2, W3
    yt = _build()(x, idx1.T, idx2.T, idx3.T)
    return yt.T


# speedup vs baseline: 1.1437x; 1.1437x over previous
import functools

import jax
import jax.numpy as jnp
from jax import lax
from jax.experimental import pallas as pl
from jax.experimental.pallas import tpu as pltpu
from jax.experimental.pallas import tpu_sc as plsc

_B = 16384
_D = 128
_ARITY = 6
_OUT = 16
_NW = 32
_RPW = _B // _NW
_L = 16
_CHUNKS = _RPW // _L
_NBLK = 2
_RPB = _RPW // _NBLK

_TWO_OVER_PI = 0.6366197723675814
_RND = 12582912.0
_PIO2_A = 1.5707963705062866
_PIO2_B = 4.37113900018624e-08


def _sincos_core(v, qoff):
    t = v * _TWO_OVER_PI
    kf = (t + _RND) - _RND
    r = (v - kf * _PIO2_A) + kf * _PIO2_B
    ki = kf.astype(jnp.int32) + qoff
    r2 = r * r
    sinp = r * (1.0 + r2 * (-1.6666667e-1 + r2 * (8.3333310e-3 + r2 * (-1.9840874e-4))))
    cosp = 1.0 + r2 * (-5.0e-1 + r2 * (4.1666638e-2 + r2 * (-1.3887316e-3)))
    p = jnp.where((ki & 1) == 1, cosp, sinp)
    return jnp.where((ki & 2) == 2, -p, p)


def _fast_sin(v):
    return _sincos_core(v, 0)


def _fast_cos(v):
    return _sincos_core(v, 1)


def _bases(g):
    return [_fast_sin(g[0]), _fast_cos(g[1]), g[2] * g[3], g[4] + g[5]]


def _sel4(c, h):
    return jnp.where(c == 0, h[0],
                     jnp.where(c == 1, h[1],
                               jnp.where(c == 2, h[2], h[3])))


@functools.cache
def _build():
    @functools.partial(
        pl.kernel,
        mesh=plsc.VectorSubcoreMesh(core_axis_name="c", subcore_axis_name="s"),
        compiler_params=pltpu.CompilerParams(
            needs_layout_passes=False, use_tc_tiling_on_sc=True),
        out_type=jax.ShapeDtypeStruct((_OUT, _B), jnp.float32),
        scratch_types=[
            pltpu.VMEM((_RPW, _D), jnp.float32),
            pltpu.VMEM((_ARITY, _RPW), jnp.int32),
            pltpu.VMEM((_ARITY, _RPW), jnp.int32),
            pltpu.VMEM((_OUT, _RPW), jnp.int32),
            pltpu.VMEM((_OUT, _RPW), jnp.float32),
            pltpu.SemaphoreType.DMA,
        ] + [pltpu.SemaphoreType.DMA] * _NBLK,
    )
    def _occam_sc(x_hbm, i1t_hbm, i2t_hbm, i3t_hbm, out_hbm,
                  x_v, i1_v, i2_v, i3_v, o_v, sem_i, *sem_x):
        wid = lax.axis_index("s") * 2 + lax.axis_index("c")
        base = wid * _RPW
        xcopies = [
            pltpu.async_copy(
                x_hbm.at[pl.ds(base + blk * _RPB, _RPB)],
                x_v.at[pl.ds(blk * _RPB, _RPB)], sem_x[blk])
            for blk in range(_NBLK)
        ]
        c1 = pltpu.async_copy(i1t_hbm.at[:, pl.ds(base, _RPW)], i1_v, sem_i)
        c2 = pltpu.async_copy(i2t_hbm.at[:, pl.ds(base, _RPW)], i2_v, sem_i)
        c3 = pltpu.async_copy(i3t_hbm.at[:, pl.ds(base, _RPW)], i3_v, sem_i)
        c1.wait(); c2.wait(); c3.wait()

        lanes = lax.iota(jnp.int32, _L)

        def chunk(c):
            rows = lanes + c * _L
            cols = pl.ds(c * _L, _L)
            g1 = [plsc.load_gather(x_v, [rows, i1_v[j, cols]])
                  for j in range(_ARITY)]
            h1 = _bases(g1)
            g2 = [_sel4(i2_v[j, cols], h1) for j in range(_ARITY)]
            h2 = _bases(g2)
            for o in range(_OUT):
                o_v[o, cols] = _sel4(i3_v[o, cols], h2)

        cpb = _RPB // _L
        for blk in range(_NBLK):
            xcopies[blk].wait()
            plsc.parallel_loop(blk * cpb, (blk + 1) * cpb, unroll=4)(chunk)
        pltpu.sync_copy(o_v, out_hbm.at[:, pl.ds(base, _RPW)])

    return _occam_sc


def kernel(x, W1, W2, W3, idx1, idx2, idx3):
    del W1, W2, W3
    yt = _build()(x, idx1.T, idx2.T, idx3.T)
    return yt.T
